# initial kernel scaffold (unmeasured)
import jax
import jax.numpy as jnp
from jax import lax
from jax.experimental import pallas as pl
from jax.experimental.pallas import tpu as pltpu

B, SQ, H, D = 4, 32, 8, 128
BH = B * H
SCALE = D ** -0.5


def kernel(Q, K, V):
    skv = K.shape[1]

    def body(q_ref, k_ref, v_ref, out_ref,
             o_self, o_rem, l_self, l_rem, send_sems, recv_sems):
        bh = pl.program_id(0)

        q = q_ref[0, :, 0, :].astype(jnp.bfloat16)
        k = k_ref[0, :, 0, :].astype(jnp.bfloat16)
        v = v_ref[0, :, 0, :].astype(jnp.bfloat16)
        s = lax.dot_general(q, k, (((1,), (1,)), ((), ())),
                            preferred_element_type=jnp.float32) * SCALE
        m = jnp.max(s, axis=1, keepdims=True)
        p = jnp.exp(s - m)
        l = jnp.sum(p, axis=1, keepdims=True)
        o = lax.dot_general(p.astype(jnp.bfloat16), v,
                            (((1,), (0,)), ((), ())),
                            preferred_element_type=jnp.float32)
        o = o / l
        lse = m + jnp.log(l)

        o_self[pl.ds(bh, 1)] = o.astype(jnp.bfloat16)[None]
        l_self[pl.ds(bh, 1)] = jnp.broadcast_to(lse, (SQ, 8))[None]

        @pl.when(bh == BH - 1)
        def _():
            my_x = lax.axis_index("x")
            my_y = lax.axis_index("y")
            my_z = lax.axis_index("z")
            partner = (1 - my_x, my_y, my_z)

            barrier = pltpu.get_barrier_semaphore()
            pl.semaphore_signal(barrier, inc=1, device_id=partner,
                                device_id_type=pl.DeviceIdType.MESH)
            pl.semaphore_wait(barrier, 1)

            rdma_o = pltpu.make_async_remote_copy(
                src_ref=o_self, dst_ref=o_rem,
                send_sem=send_sems.at[0], recv_sem=recv_sems.at[0],
                device_id=partner, device_id_type=pl.DeviceIdType.MESH)
            rdma_l = pltpu.make_async_remote_copy(
                src_ref=l_self, dst_ref=l_rem,
                send_sem=send_sems.at[1], recv_sem=recv_sems.at[1],
                device_id=partner, device_id_type=pl.DeviceIdType.MESH)
            rdma_o.start()
            rdma_l.start()
            rdma_o.wait()
            rdma_l.wait()

            for i in range(BH):
                b, h = divmod(i, H)
                ls = l_self[i, :, 0:1]
                lr = l_rem[i, :, 0:1]
                os_ = o_self[i].astype(jnp.float32)
                or_ = o_rem[i].astype(jnp.float32)
                w = 1.0 / (1.0 + jnp.exp(ls - lr))
                out_ref[b, :, h, :] = os_ + w * (or_ - os_)

    return pl.pallas_call(
        body,
        grid=(BH,),
        in_specs=[
            pl.BlockSpec((1, SQ, 1, D), lambda i: (i // H, 0, i % H, 0)),
            pl.BlockSpec((1, skv, 1, D), lambda i: (i // H, 0, i % H, 0)),
            pl.BlockSpec((1, skv, 1, D), lambda i: (i // H, 0, i % H, 0)),
        ],
        out_specs=pl.BlockSpec((B, SQ, H, D), lambda i: (0, 0, 0, 0)),
        out_shape=jax.ShapeDtypeStruct((B, SQ, H, D), jnp.float32),
        scratch_shapes=[
            pltpu.VMEM((BH, SQ, D), jnp.bfloat16),
            pltpu.VMEM((BH, SQ, D), jnp.bfloat16),
            pltpu.VMEM((BH, SQ, 8), jnp.float32),
            pltpu.VMEM((BH, SQ, 8), jnp.float32),
            pltpu.SemaphoreType.DMA((2,)),
            pltpu.SemaphoreType.DMA((2,)),
        ],
        compiler_params=pltpu.CompilerParams(collective_id=0),
    )(Q, K, V)


# baseline (device time: 189217 ns/iter reference)
import jax
import jax.numpy as jnp
from jax import lax
from jax.experimental import pallas as pl
from jax.experimental.pallas import tpu as pltpu

B, SQ, H, D = 4, 32, 8, 128
BH = B * H
SCALE = D ** -0.5
CK = 1024
NEG_BIG = -1e30


def kernel(Q, K, V):
    skv = K.shape[1]
    nc = skv // CK

    q_t = Q.transpose(0, 2, 1, 3).reshape(BH, SQ, D)

    def body(q_ref, k_ref, v_ref, out_ref,
             m_run, l_run, acc_run,
             o_self, o_rem, lse_self, lse_rem,
             send_sems, recv_sems):
        b = pl.program_id(0)
        c = pl.program_id(1)

        @pl.when(c == 0)
        def _():
            m_run[...] = jnp.full((H, SQ, 128), NEG_BIG, jnp.float32)
            l_run[...] = jnp.zeros((H, SQ, 128), jnp.float32)
            acc_run[...] = jnp.zeros((H, SQ, D), jnp.float32)

        for h in range(H):
            q = q_ref[h].astype(jnp.bfloat16)
            k = k_ref[0, :, h, :].astype(jnp.bfloat16)
            v = v_ref[0, :, h, :].astype(jnp.bfloat16)
            s = lax.dot_general(q, k, (((1,), (1,)), ((), ())),
                                preferred_element_type=jnp.float32) * SCALE
            m_c = jnp.max(s, axis=1, keepdims=True)
            m_old = m_run[h][:, 0:1]
            m_new = jnp.maximum(m_old, m_c)
            alpha = jnp.exp(m_old - m_new)
            p = jnp.exp(s - m_new)
            l_new = alpha * l_run[h][:, 0:1] + jnp.sum(p, axis=1, keepdims=True)
            pv = lax.dot_general(p.astype(jnp.bfloat16), v,
                                 (((1,), (0,)), ((), ())),
                                 preferred_element_type=jnp.float32)
            acc_new = alpha * acc_run[h] + pv
            m_run[h] = jnp.broadcast_to(m_new, (SQ, 128))
            l_run[h] = jnp.broadcast_to(l_new, (SQ, 128))
            acc_run[h] = acc_new

            @pl.when(c == nc - 1)
            def _():
                o = (acc_new / l_new).astype(jnp.bfloat16)
                lse = m_new + jnp.log(l_new)
                o_self[pl.ds(b * H + h, 1)] = o[None]
                lse_self[pl.ds(b * H + h, 1)] = jnp.broadcast_to(lse, (SQ, 8))[None]

        @pl.when((b == B - 1) & (c == nc - 1))
        def _():
            my_x = lax.axis_index("x")
            my_y = lax.axis_index("y")
            my_z = lax.axis_index("z")
            partner = (1 - my_x, my_y, my_z)

            barrier = pltpu.get_barrier_semaphore()
            pl.semaphore_signal(barrier, inc=1, device_id=partner,
                                device_id_type=pl.DeviceIdType.MESH)
            pl.semaphore_wait(barrier, 1)

            rdma_o = pltpu.make_async_remote_copy(
                src_ref=o_self, dst_ref=o_rem,
                send_sem=send_sems.at[0], recv_sem=recv_sems.at[0],
                device_id=partner, device_id_type=pl.DeviceIdType.MESH)
            rdma_l = pltpu.make_async_remote_copy(
                src_ref=lse_self, dst_ref=lse_rem,
                send_sem=send_sems.at[1], recv_sem=recv_sems.at[1],
                device_id=partner, device_id_type=pl.DeviceIdType.MESH)
            rdma_o.start()
            rdma_l.start()
            rdma_o.wait()
            rdma_l.wait()

            for j in range(BH):
                ls = lse_self[j][:, 0:1]
                lr = lse_rem[j][:, 0:1]
                os_ = o_self[j].astype(jnp.float32)
                or_ = o_rem[j].astype(jnp.float32)
                w = 1.0 / (1.0 + jnp.exp(ls - lr))
                out_ref[j] = os_ + w * (or_ - os_)

    out = pl.pallas_call(
        body,
        grid=(B, nc),
        in_specs=[
            pl.BlockSpec((H, SQ, D), lambda b, c: (b, 0, 0)),
            pl.BlockSpec((1, CK, H, D), lambda b, c: (b, c, 0, 0)),
            pl.BlockSpec((1, CK, H, D), lambda b, c: (b, c, 0, 0)),
        ],
        out_specs=pl.BlockSpec((BH, SQ, D), lambda b, c: (0, 0, 0)),
        out_shape=jax.ShapeDtypeStruct((BH, SQ, D), jnp.float32),
        scratch_shapes=[
            pltpu.VMEM((H, SQ, 128), jnp.float32),
            pltpu.VMEM((H, SQ, 128), jnp.float32),
            pltpu.VMEM((H, SQ, D), jnp.float32),
            pltpu.VMEM((BH, SQ, D), jnp.bfloat16),
            pltpu.VMEM((BH, SQ, D), jnp.bfloat16),
            pltpu.VMEM((BH, SQ, 8), jnp.float32),
            pltpu.VMEM((BH, SQ, 8), jnp.float32),
            pltpu.SemaphoreType.DMA((2,)),
            pltpu.SemaphoreType.DMA((2,)),
        ],
        compiler_params=pltpu.CompilerParams(collective_id=0),
    )(q_t, K, V)

    return out.reshape(B, H, SQ, D).transpose(0, 2, 1, 3)


# device time: 180367 ns/iter; 1.0491x vs baseline; 1.0491x over previous
import jax
import jax.numpy as jnp
from jax import lax
from jax.experimental import pallas as pl
from jax.experimental.pallas import tpu as pltpu

B, SQ, H, D = 4, 32, 8, 128
BH = B * H
SCALE = D ** -0.5
CK = 1024
NEG_BIG = -1e30


def kernel(Q, K, V):
    skv = K.shape[1]
    nc = skv // CK

    q_t = Q.transpose(0, 2, 1, 3).reshape(BH, SQ, D)

    def body(q_ref, k_ref, v_ref, out_ref,
             m_run, l_run, acc_run,
             kh_buf, vh_buf,
             o_self, o_rem, lse_self, lse_rem,
             kc_sems, vc_sems, send_sems, recv_sems):
        b = pl.program_id(0)
        c = pl.program_id(1)

        @pl.when(c == 0)
        def _():
            m_run[...] = jnp.full((H, SQ, 128), NEG_BIG, jnp.float32)
            l_run[...] = jnp.zeros((H, SQ, 128), jnp.float32)
            acc_run[...] = jnp.zeros((H, SQ, D), jnp.float32)

        for h in range(H):
            pltpu.make_async_copy(
                k_ref.at[0, :, h, :], kh_buf.at[h], kc_sems.at[h]).start()
            pltpu.make_async_copy(
                v_ref.at[0, :, h, :], vh_buf.at[h], vc_sems.at[h]).start()

        for h in range(H):
            pltpu.make_async_copy(
                k_ref.at[0, :, h, :], kh_buf.at[h], kc_sems.at[h]).wait()
            pltpu.make_async_copy(
                v_ref.at[0, :, h, :], vh_buf.at[h], vc_sems.at[h]).wait()
            q = q_ref[h].astype(jnp.bfloat16)
            k = kh_buf[h].astype(jnp.bfloat16)
            v = vh_buf[h].astype(jnp.bfloat16)
            s = lax.dot_general(q, k, (((1,), (1,)), ((), ())),
                                preferred_element_type=jnp.float32) * SCALE
            m_c = jnp.max(s, axis=1, keepdims=True)
            m_old = m_run[h][:, 0:1]
            m_new = jnp.maximum(m_old, m_c)
            alpha = jnp.exp(m_old - m_new)
            p = jnp.exp(s - m_new)
            l_new = alpha * l_run[h][:, 0:1] + jnp.sum(p, axis=1, keepdims=True)
            pv = lax.dot_general(p.astype(jnp.bfloat16), v,
                                 (((1,), (0,)), ((), ())),
                                 preferred_element_type=jnp.float32)
            acc_new = alpha * acc_run[h] + pv
            m_run[h] = jnp.broadcast_to(m_new, (SQ, 128))
            l_run[h] = jnp.broadcast_to(l_new, (SQ, 128))
            acc_run[h] = acc_new

            @pl.when(c == nc - 1)
            def _():
                o = (acc_new / l_new).astype(jnp.bfloat16)
                lse = m_new + jnp.log(l_new)
                o_self[pl.ds(b * H + h, 1)] = o[None]
                lse_self[pl.ds(b * H + h, 1)] = jnp.broadcast_to(lse, (SQ, 8))[None]

        @pl.when((b == B - 1) & (c == nc - 1))
        def _():
            my_x = lax.axis_index("x")
            my_y = lax.axis_index("y")
            my_z = lax.axis_index("z")
            partner = (1 - my_x, my_y, my_z)

            barrier = pltpu.get_barrier_semaphore()
            pl.semaphore_signal(barrier, inc=1, device_id=partner,
                                device_id_type=pl.DeviceIdType.MESH)
            pl.semaphore_wait(barrier, 1)

            rdma_o = pltpu.make_async_remote_copy(
                src_ref=o_self, dst_ref=o_rem,
                send_sem=send_sems.at[0], recv_sem=recv_sems.at[0],
                device_id=partner, device_id_type=pl.DeviceIdType.MESH)
            rdma_l = pltpu.make_async_remote_copy(
                src_ref=lse_self, dst_ref=lse_rem,
                send_sem=send_sems.at[1], recv_sem=recv_sems.at[1],
                device_id=partner, device_id_type=pl.DeviceIdType.MESH)
            rdma_o.start()
            rdma_l.start()
            rdma_o.wait()
            rdma_l.wait()

            for j in range(BH):
                ls = lse_self[j][:, 0:1]
                lr = lse_rem[j][:, 0:1]
                os_ = o_self[j].astype(jnp.float32)
                or_ = o_rem[j].astype(jnp.float32)
                w = 1.0 / (1.0 + jnp.exp(ls - lr))
                out_ref[j] = os_ + w * (or_ - os_)

    out = pl.pallas_call(
        body,
        grid=(B, nc),
        in_specs=[
            pl.BlockSpec((H, SQ, D), lambda b, c: (b, 0, 0)),
            pl.BlockSpec((1, CK, H, D), lambda b, c: (b, c, 0, 0)),
            pl.BlockSpec((1, CK, H, D), lambda b, c: (b, c, 0, 0)),
        ],
        out_specs=pl.BlockSpec((BH, SQ, D), lambda b, c: (0, 0, 0)),
        out_shape=jax.ShapeDtypeStruct((BH, SQ, D), jnp.float32),
        scratch_shapes=[
            pltpu.VMEM((H, SQ, 128), jnp.float32),
            pltpu.VMEM((H, SQ, 128), jnp.float32),
            pltpu.VMEM((H, SQ, D), jnp.float32),
            pltpu.VMEM((H, CK, D), jnp.float32),
            pltpu.VMEM((H, CK, D), jnp.float32),
            pltpu.VMEM((BH, SQ, D), jnp.bfloat16),
            pltpu.VMEM((BH, SQ, D), jnp.bfloat16),
            pltpu.VMEM((BH, SQ, 8), jnp.float32),
            pltpu.VMEM((BH, SQ, 8), jnp.float32),
            pltpu.SemaphoreType.DMA((H,)),
            pltpu.SemaphoreType.DMA((H,)),
            pltpu.SemaphoreType.DMA((2,)),
            pltpu.SemaphoreType.DMA((2,)),
        ],
        compiler_params=pltpu.CompilerParams(collective_id=0),
    )(q_t, K, V)

    return out.reshape(B, H, SQ, D).transpose(0, 2, 1, 3)


# device time: 42662 ns/iter; 4.4353x vs baseline; 4.2278x over previous
import functools
import jax
import jax.numpy as jnp
from jax import lax
from jax.experimental import pallas as pl
from jax.experimental.pallas import tpu as pltpu

B, SQ, H, D = 4, 32, 8, 128
BH = B * H
HALF = BH // 2
SCALE = D ** -0.5
NREP = 8
NROUND = 4
LOOKAHEAD = 8


def kernel(Q, K, V):
    skv = K.shape[1]
    sk = skv // NREP

    q_t = Q.transpose(0, 2, 1, 3).reshape(BH, SQ, D)

    def body(q_ref, k_ref, v_ref, out_ref,
             kh, vh, o_cur, lse_cur, o_send, o_rem, lse_rem,
             kc_sems, vc_sems,
             o_send_sems, l_send_sems, o_recv_sems, l_recv_sems):
        my_x = lax.axis_index("x")
        my_y = lax.axis_index("y")
        my_z = lax.axis_index("z")
        r = my_y * 4 + my_z
        base = r * sk

        partners = [
            (my_x, my_y, my_z ^ 2),
            (1 - my_x, my_y, my_z),
            (my_x, 1 - my_y, my_z),
            (my_x, my_y, my_z ^ 1),
        ]

        barrier = pltpu.get_barrier_semaphore()
        for p_id in partners:
            pl.semaphore_signal(barrier, inc=1, device_id=p_id,
                                device_id_type=pl.DeviceIdType.MESH)
        pl.semaphore_wait(barrier, NROUND)

        def start_dma(j):
            b, h = divmod(j, H)
            pltpu.make_async_copy(
                k_ref.at[b, pl.ds(base, sk), h, :], kh.at[j],
                kc_sems.at[j]).start()
            pltpu.make_async_copy(
                v_ref.at[b, pl.ds(base, sk), h, :], vh.at[j],
                vc_sems.at[j]).start()

        for j in range(LOOKAHEAD):
            start_dma(j)

        lse_cur[...] = jnp.zeros((SQ, 128), jnp.float32)

        def local_partial(j):
            pltpu.make_async_copy(
                k_ref.at[0, pl.ds(base, sk), 0, :], kh.at[j],
                kc_sems.at[j]).wait()
            pltpu.make_async_copy(
                v_ref.at[0, pl.ds(base, sk), 0, :], vh.at[j],
                vc_sems.at[j]).wait()
            if j + LOOKAHEAD < BH:
                start_dma(j + LOOKAHEAD)
            q = q_ref[j].astype(jnp.bfloat16)
            k = kh[j].astype(jnp.bfloat16)
            v = vh[j].astype(jnp.bfloat16)
            s = lax.dot_general(q, k, (((1,), (1,)), ((), ())),
                                preferred_element_type=jnp.float32) * SCALE
            m = jnp.max(s, axis=1, keepdims=True)
            p = jnp.exp(s - m)
            l = jnp.sum(p, axis=1, keepdims=True)
            pv = lax.dot_general(p.astype(jnp.bfloat16), v,
                                 (((1,), (0,)), ((), ())),
                                 preferred_element_type=jnp.float32)
            o_cur[pl.ds(j, 1)] = (pv / l)[None]
            lse_cur[:, j:j + 1] = m + jnp.log(l)

        def make_o_rdma(rd, lo, n, slot):
            return pltpu.make_async_remote_copy(
                src_ref=o_send.at[pl.ds(lo, n)],
                dst_ref=o_rem.at[rd, pl.ds(lo, n)],
                send_sem=o_send_sems.at[slot], recv_sem=o_recv_sems.at[slot],
                device_id=partners[rd],
                device_id_type=pl.DeviceIdType.MESH)

        def make_l_rdma(rd):
            return pltpu.make_async_remote_copy(
                src_ref=lse_cur, dst_ref=lse_rem.at[rd],
                send_sem=l_send_sems.at[rd], recv_sem=l_recv_sems.at[rd],
                device_id=partners[rd],
                device_id_type=pl.DeviceIdType.MESH)

        def combine(rd, weights, lo, n):
            for j in range(lo, lo + n):
                w = weights[:, j:j + 1]
                oc = o_cur[j]
                o_cur[pl.ds(j, 1)] = (oc + w * (
                    o_rem[rd, j].astype(jnp.float32) - oc))[None]

        for j in range(HALF):
            local_partial(j)
        o_send[pl.ds(0, HALF)] = o_cur[0:HALF].astype(jnp.bfloat16)
        rdma_0a = make_o_rdma(0, 0, HALF, 0)
        rdma_0a.start()

        for j in range(HALF, BH):
            local_partial(j)
        o_send[pl.ds(HALF, HALF)] = o_cur[HALF:BH].astype(jnp.bfloat16)
        rdma_0b = make_o_rdma(0, HALF, HALF, NROUND)
        rdma_0b.start()
        rdma_l0 = make_l_rdma(0)
        rdma_l0.start()

        rdma_l0.wait()
        ls = lse_cur[...]
        lr = lse_rem[0]
        w0 = 1.0 / (1.0 + jnp.exp(ls - lr))
        mx = jnp.maximum(ls, lr)
        lse_cur[...] = mx + jnp.log(jnp.exp(ls - mx) + jnp.exp(lr - mx))
        rdma_0a.wait()
        combine(0, w0, 0, HALF)
        rdma_0b.wait()
        combine(0, w0, HALF, HALF)

        for rd in range(1, NROUND):
            o_send[...] = o_cur[...].astype(jnp.bfloat16)
            rdma_o = make_o_rdma(rd, 0, BH, rd)
            rdma_l = make_l_rdma(rd)
            rdma_o.start()
            rdma_l.start()

            rdma_l.wait()
            ls = lse_cur[...]
            lr = lse_rem[rd]
            w = 1.0 / (1.0 + jnp.exp(ls - lr))
            mx = jnp.maximum(ls, lr)
            lse_cur[...] = mx + jnp.log(jnp.exp(ls - mx) + jnp.exp(lr - mx))
            rdma_o.wait()
            combine(rd, w, 0, BH)

        out_ref[...] = o_cur[...]

        @functools.partial(pl.run_scoped,
                           second_barrier=pltpu.SemaphoreType.REGULAR)
        def _(second_barrier):
            for p_id in partners:
                pl.semaphore_signal(second_barrier, inc=1, device_id=p_id,
                                    device_id_type=pl.DeviceIdType.MESH)
            pl.semaphore_wait(second_barrier, NROUND)

    out = pl.pallas_call(
        body,
        grid=(1,),
        in_specs=[
            pl.BlockSpec((BH, SQ, D), lambda i: (0, 0, 0)),
            pl.BlockSpec(memory_space=pl.ANY),
            pl.BlockSpec(memory_space=pl.ANY),
        ],
        out_specs=pl.BlockSpec((BH, SQ, D), lambda i: (0, 0, 0)),
        out_shape=jax.ShapeDtypeStruct((BH, SQ, D), jnp.float32),
        scratch_shapes=[
            pltpu.VMEM((BH, sk, D), jnp.float32),
            pltpu.VMEM((BH, sk, D), jnp.float32),
            pltpu.VMEM((BH, SQ, D), jnp.float32),
            pltpu.VMEM((SQ, 128), jnp.float32),
            pltpu.VMEM((BH, SQ, D), jnp.bfloat16),
            pltpu.VMEM((NROUND, BH, SQ, D), jnp.bfloat16),
            pltpu.VMEM((NROUND, SQ, 128), jnp.float32),
            pltpu.SemaphoreType.DMA((BH,)),
            pltpu.SemaphoreType.DMA((BH,)),
            pltpu.SemaphoreType.DMA((NROUND + 1,)),
            pltpu.SemaphoreType.DMA((NROUND,)),
            pltpu.SemaphoreType.DMA((NROUND + 1,)),
            pltpu.SemaphoreType.DMA((NROUND,)),
        ],
        compiler_params=pltpu.CompilerParams(collective_id=0),
    )(q_t, K, V)

    return out.reshape(B, H, SQ, D).transpose(0, 2, 1, 3)
